# baseline (speedup 1.0000x reference)
import math

import jax
import jax.numpy as jnp
from jax.experimental import pallas as pl
from jax.experimental.pallas import tpu as pltpu


def _mlp_kernel(x_ref, w1_ref, b1_ref, w2_ref, b2_ref, w3_ref, b3_ref, o_ref):
    x = x_ref[...]
    h1 = jax.nn.sigmoid(
        jnp.dot(w1_ref[...], x, preferred_element_type=jnp.float32)
        + b1_ref[...])
    h2 = jax.nn.sigmoid(
        jnp.dot(w2_ref[...], h1, preferred_element_type=jnp.float32)
        + b2_ref[...])
    w3 = w3_ref[...]
    b3 = b3_ref[...]
    d = (jnp.dot(w3[1:2, :] - w3[0:1, :], h2,
                 preferred_element_type=jnp.float32)
         + (b3[1:2, :] - b3[0:1, :]))
    p1 = jax.nn.sigmoid(d)
    o_ref[...] = jnp.concatenate([1.0 - p1, p1], axis=0)


@jax.jit
def kernel(x, w1, b1, w2, b2, w3, b3):
    B, F = x.shape
    f32 = jnp.float32
    xT = x.astype(f32).T

    tb = math.gcd(B, 131072)
    grid = B // tb
    full = lambda shape: pl.BlockSpec(shape, lambda i: (0, 0))

    out = pl.pallas_call(
        _mlp_kernel,
        out_shape=jax.ShapeDtypeStruct((2, B), f32),
        grid=(grid,),
        in_specs=[
            pl.BlockSpec((F, tb), lambda i: (0, i)),
            full((12, 12)), full((12, 1)),
            full((8, 12)), full((8, 1)),
            full((2, 8)), full((2, 1)),
        ],
        out_specs=pl.BlockSpec((2, tb), lambda i: (0, i)),
        compiler_params=pltpu.CompilerParams(
            dimension_semantics=("parallel",)),
    )(xT, w1.astype(f32), b1.astype(f32), w2.astype(f32), b2.astype(f32),
      w3.astype(f32), b3.astype(f32))

    return out.T
